# baseline (speedup 1.0000x reference)
import jax
import jax.numpy as jnp
from jax.experimental import pallas as pl
from jax.experimental.pallas import tpu as pltpu

_N = 4096
_D = 64
_L = 128
_K = 192
_S = 1024.0
_BIG = _S * _S
_MARGIN = 1.0
_BM = 512
_NB = _N // _BM


def _triplet_kernel(e_all_ref, t_all_ref, out_ref, a_ref, b_ref):
    ef = e_all_ref[...]
    tj = t_all_ref[...]
    lanes = jax.lax.broadcasted_iota(jnp.int32, (1, _L), 1)
    oh = (tj == lanes).astype(jnp.float32)
    sq = jnp.sum(ef * ef, axis=1, keepdims=True)
    ohs = (oh * _S).astype(jnp.bfloat16)
    sqh = sq.astype(jnp.bfloat16)
    oneh = jnp.ones((_N, 1), jnp.bfloat16)
    zfill = jnp.zeros((_N, _L - 102), jnp.bfloat16)

    a_ref[:, 0:_L] = ohs
    a_ref[:, 100:101] = oneh
    a_ref[:, 101:102] = sqh
    a_ref[:, 102:_L] = zfill
    a_ref[:, _L:_K] = (ef * -2.0).astype(jnp.bfloat16)

    b_ref[:, 0:_L] = ohs
    b_ref[:, 100:101] = sqh
    b_ref[:, 101:102] = oneh
    b_ref[:, 102:_L] = zfill
    b_ref[:, _L:_K] = ef.astype(jnp.bfloat16)

    hist = jnp.sum(oh, axis=0, keepdims=True)
    cnt = jnp.sum(oh * hist, axis=1, keepdims=True)
    t0 = t_all_ref[0, 0]
    d0corr = jnp.where(tj == t0, _BIG, 0.0)

    rmax = [None] * _NB
    rmin = [None] * _NB
    d0row = [None] * _NB

    def comb(cur, val, op):
        return val if cur is None else op(cur, val)

    for i in range(_NB):
        for j in range(i, _NB):
            c = jax.lax.dot_general(
                a_ref[pl.ds(i * _BM, _BM), :], b_ref[pl.ds(j * _BM, _BM), :],
                (((1,), (1,)), ((), ())),
                preferred_element_type=jnp.float32)
            rmax[i] = comb(rmax[i], jnp.max(c, axis=1), jnp.maximum)
            rmin[i] = comb(rmin[i], jnp.min(c, axis=1), jnp.minimum)
            if j > i:
                rmax[j] = comb(rmax[j], jnp.max(c, axis=0), jnp.maximum)
                rmin[j] = comb(rmin[j], jnp.min(c, axis=0), jnp.minimum)
            if i == 0:
                d0row[j] = c[0, :]

    total = jnp.zeros((), jnp.float32)
    for k in range(_NB):
        rows = slice(k * _BM, (k + 1) * _BM)
        pos_v = rmax[k] - _BIG
        neg_v = rmin[k]
        d0 = d0row[k] - d0corr[rows, 0]
        count = cnt[rows, 0]
        ap = jnp.where(count > 1.5, pos_v, d0)
        an = jnp.where(count < _N - 0.5, neg_v, d0)
        total = total + jnp.sum(jnp.maximum(ap - an + _MARGIN, 0.0))

    out_ref[...] = (total / _N).reshape(1, 1, 1)


def _triplet_mean_loss(embeddings, target):
    tcol = target.astype(jnp.int32).reshape(_N, 1)
    out = pl.pallas_call(
        _triplet_kernel,
        in_specs=[
            pl.BlockSpec((_N, _D), lambda: (0, 0)),
            pl.BlockSpec((_N, 1), lambda: (0, 0)),
        ],
        out_specs=pl.BlockSpec((1, 1, 1), lambda: (0, 0, 0)),
        out_shape=jax.ShapeDtypeStruct((1, 1, 1), jnp.float32),
        scratch_shapes=[
            pltpu.VMEM((_N, _K), jnp.bfloat16),
            pltpu.VMEM((_N, _K), jnp.bfloat16),
        ],
    )(embeddings, tcol)
    return out.reshape(())


def kernel(embeddings, target):
    return (_triplet_mean_loss(embeddings, target), _N)


# speedup vs baseline: 1.0278x; 1.0278x over previous
import jax
import jax.numpy as jnp
from jax.experimental import pallas as pl
from jax.experimental.pallas import tpu as pltpu

_N = 4096
_D = 64
_L = 128
_K = 256
_S = 1024.0
_BIG = _S * _S
_MARGIN = 1.0
_BM = 512


def _triplet_kernel(e_all_ref, t_all_ref, out_ref, a_ref, b_ref):
    ef = e_all_ref[...]
    tj = t_all_ref[...]
    lanes = jax.lax.broadcasted_iota(jnp.int32, (1, _L), 1)
    oh = (tj == lanes).astype(jnp.float32)
    sq = jnp.sum(ef * ef, axis=1, keepdims=True)
    ohs = (oh * _S).astype(jnp.bfloat16)
    zpad = jnp.zeros((_N, _L - _D - 1), jnp.bfloat16)

    a_ref[:, 0:_D] = (ef * -2.0).astype(jnp.bfloat16)
    a_ref[:, _D:_D + 1] = jnp.ones((_N, 1), jnp.bfloat16)
    a_ref[:, _D + 1:_L] = zpad
    a_ref[:, _L:_K] = ohs

    b_ref[:, 0:_D] = ef.astype(jnp.bfloat16)
    b_ref[:, _D:_D + 1] = sq.astype(jnp.bfloat16)
    b_ref[:, _D + 1:_L] = zpad
    b_ref[:, _L:_K] = ohs

    hist = jnp.sum(oh, axis=0, keepdims=True)
    cnt = jnp.sum(oh * hist, axis=1, keepdims=True)
    t0 = t_all_ref[0, 0]
    d0corr = jnp.where(tj == t0, _BIG, 0.0)

    nb = _N // _BM
    total = jnp.zeros((), jnp.float32)
    for blk in range(nb):
        rows = pl.ds(blk * _BM, _BM)
        c = jax.lax.dot_general(
            a_ref[rows, :], b_ref[...], (((1,), (1,)), ((), ())),
            preferred_element_type=jnp.float32)
        sq_i = sq[blk * _BM:(blk + 1) * _BM, 0]
        pos_v = jnp.max(c, axis=1) + sq_i - _BIG
        neg_v = jnp.min(c, axis=1) + sq_i
        d0 = c[:, 0] + sq_i - d0corr[blk * _BM:(blk + 1) * _BM, 0]
        count = cnt[blk * _BM:(blk + 1) * _BM, 0]
        ap = jnp.where(count > 1.5, pos_v, d0)
        an = jnp.where(count < _N - 0.5, neg_v, d0)
        total = total + jnp.sum(jnp.maximum(ap - an + _MARGIN, 0.0))

    out_ref[...] = (total / _N).reshape(1, 1, 1)


def _triplet_mean_loss(embeddings, target):
    tcol = target.astype(jnp.int32).reshape(_N, 1)
    out = pl.pallas_call(
        _triplet_kernel,
        in_specs=[
            pl.BlockSpec((_N, _D), lambda: (0, 0)),
            pl.BlockSpec((_N, 1), lambda: (0, 0)),
        ],
        out_specs=pl.BlockSpec((1, 1, 1), lambda: (0, 0, 0)),
        out_shape=jax.ShapeDtypeStruct((1, 1, 1), jnp.float32),
        scratch_shapes=[
            pltpu.VMEM((_N, _K), jnp.bfloat16),
            pltpu.VMEM((_N, _K), jnp.bfloat16),
        ],
    )(embeddings, tcol)
    return out.reshape(())


def kernel(embeddings, target):
    return (_triplet_mean_loss(embeddings, target), _N)
